# Initial kernel scaffold; baseline (speedup 1.0000x reference)
import jax
import jax.numpy as jnp
from jax.experimental import pallas as pl


def kernel(logits, position_ids):
    raise NotImplementedError("write your pallas kernel here")


# speedup vs baseline: 3.3893x; 3.3893x over previous
import jax
import jax.numpy as jnp
from jax.experimental import pallas as pl
from jax.experimental.pallas import tpu as pltpu

K = 50
N = 1_000_000
R, C = 8, 125_000

def _body(x_ref, o_ref):
    _INT_MIN = jnp.int32(-(2**31))
    x = x_ref[...]
    ib = jax.lax.bitcast_convert_type(x, jnp.int32)
    key = ib ^ ((ib >> 31) & jnp.int32(0x7FFFFFFF))

    def vstep(t, T):
        b = 31 - t
        trial = T | (jnp.int32(1) << b)
        cnt = jnp.sum((key >= (trial ^ _INT_MIN)).astype(jnp.int32))
        return jnp.where(cnt >= K, trial, T)

    T = jax.lax.fori_loop(0, 32, vstep, jnp.int32(0))
    kth = T ^ _INT_MIN

    c_gt = jnp.sum((key > kth).astype(jnp.int32))
    need = K - c_gt
    eq = key == kth
    row = jax.lax.broadcasted_iota(jnp.int32, (R, C), 0)
    col = jax.lax.broadcasted_iota(jnp.int32, (R, C), 1)
    idx = row * C + col

    def istep(t, res):
        b = 19 - t
        trial = res | (jnp.int32(1) << b)
        cnt = jnp.sum((eq & (idx < trial)).astype(jnp.int32))
        return jnp.where(cnt < need, trial, res)

    res = jax.lax.fori_loop(0, 20, istep, jnp.int32(0))

    keep = (key > kth) | (eq & (idx <= res))
    o_ref[...] = jnp.where(keep, x, jnp.float32(-jnp.inf))


def kernel(logits, position_ids):
    del position_ids
    x = logits.reshape(R, C)
    out = pl.pallas_call(
        _body,
        out_shape=jax.ShapeDtypeStruct((R, C), jnp.float32),
    )(x)
    return out.reshape(1, N)


# speedup vs baseline: 5.7557x; 1.6982x over previous
import jax
import jax.numpy as jnp
from jax.experimental import pallas as pl
from jax.experimental.pallas import tpu as pltpu

K = 50
N = 1_000_000
R, C = 1000, 1000


def _key_of(x):
    ib = jax.lax.bitcast_convert_type(x, jnp.int32)
    return ib ^ ((ib >> 31) & jnp.int32(0x7FFFFFFF))


def _body(x_ref, o_ref):
    _INT_MIN = jnp.int32(-(2**31))

    def fold(i, cs):
        v = x_ref[pl.ds(i * 8, 8), :]
        c0, c1, c2, c3 = cs
        m = jnp.maximum(c0, v); v = jnp.minimum(c0, v); c0 = m
        m = jnp.maximum(c1, v); v = jnp.minimum(c1, v); c1 = m
        m = jnp.maximum(c2, v); v = jnp.minimum(c2, v); c2 = m
        c3 = jnp.maximum(c3, v)
        return (c0, c1, c2, c3)

    neg = jnp.full((8, C), -jnp.inf, jnp.float32)
    cands = jax.lax.fori_loop(0, R // 8, fold, (neg, neg, neg, neg))
    kc = [_key_of(c) for c in cands]

    def cstep(t, T):
        b = 31 - t
        trial = T | (jnp.int32(1) << b)
        thr = trial ^ _INT_MIN
        cnt = (jnp.sum((kc[0] >= thr).astype(jnp.int32))
               + jnp.sum((kc[1] >= thr).astype(jnp.int32))
               + jnp.sum((kc[2] >= thr).astype(jnp.int32))
               + jnp.sum((kc[3] >= thr).astype(jnp.int32)))
        return jnp.where(cnt >= K, trial, T)

    Tc = jax.lax.fori_loop(0, 32, cstep, jnp.int32(0))
    kth_cand = Tc ^ _INT_MIN

    x = x_ref[...]
    key = _key_of(x)
    c_gt_cand = jnp.sum((key > kth_cand).astype(jnp.int32))

    def full_path():
        def vstep(t, T):
            b = 31 - t
            trial = T | (jnp.int32(1) << b)
            cnt = jnp.sum((key >= (trial ^ _INT_MIN)).astype(jnp.int32))
            return jnp.where(cnt >= K, trial, T)

        T = jax.lax.fori_loop(0, 32, vstep, jnp.int32(0))
        kth_f = T ^ _INT_MIN
        return kth_f, jnp.sum((key > kth_f).astype(jnp.int32))

    kth, c_gt = jax.lax.cond(
        c_gt_cand < K, lambda: (kth_cand, c_gt_cand), full_path)

    eq = key == kth
    t_eq = jnp.sum(eq.astype(jnp.int32))
    need = K - c_gt

    def simple_keep():
        return jnp.where(key >= kth, x, jnp.float32(-jnp.inf))

    def tie_keep():
        row = jax.lax.broadcasted_iota(jnp.int32, (R, C), 0)
        col = jax.lax.broadcasted_iota(jnp.int32, (R, C), 1)
        idx = row * C + col

        def istep(t, res):
            b = 19 - t
            trial = res | (jnp.int32(1) << b)
            cnt = jnp.sum((eq & (idx < trial)).astype(jnp.int32))
            return jnp.where(cnt < need, trial, res)

        res = jax.lax.fori_loop(0, 20, istep, jnp.int32(0))
        keep = (key > kth) | (eq & (idx <= res))
        return jnp.where(keep, x, jnp.float32(-jnp.inf))

    o_ref[...] = jax.lax.cond(need == t_eq, simple_keep, tie_keep)


def kernel(logits, position_ids):
    del position_ids
    x = logits.reshape(R, C)
    out = pl.pallas_call(
        _body,
        out_shape=jax.ShapeDtypeStruct((R, C), jnp.float32),
    )(x)
    return out.reshape(1, N)


# speedup vs baseline: 5.8418x; 1.0150x over previous
import jax
import jax.numpy as jnp
from jax.experimental import pallas as pl
from jax.experimental.pallas import tpu as pltpu

K = 50
N = 1_000_000
R, C = 1000, 1000


def _key_of(x):
    ib = jax.lax.bitcast_convert_type(x, jnp.int32)
    return ib ^ ((ib >> 31) & jnp.int32(0x7FFFFFFF))


def _body(x_ref, o_ref):
    _INT_MIN = jnp.int32(-(2**31))

    def fold(i, cs):
        v = x_ref[pl.ds(i * 8, 8), :]
        c0, c1 = cs
        m = jnp.maximum(c0, v); v = jnp.minimum(c0, v); c0 = m
        c1 = jnp.maximum(c1, v)
        return (c0, c1)

    neg = jnp.full((8, C), -jnp.inf, jnp.float32)
    c0, c1 = jax.lax.fori_loop(0, R // 8, fold, (neg, neg))
    kcand = _key_of(jnp.concatenate([c0, c1], axis=0))

    def cstep(t, T):
        b = 31 - t
        trial = T | (jnp.int32(1) << b)
        thr = trial ^ _INT_MIN
        cnt = jnp.sum((kcand >= thr).astype(jnp.int32))
        return jnp.where(cnt >= K, trial, T)

    Tc = jax.lax.fori_loop(0, 32, cstep, jnp.int32(0))
    kth_cand = Tc ^ _INT_MIN

    x = x_ref[...]
    key = _key_of(x)
    c_gt_cand = jnp.sum((key > kth_cand).astype(jnp.int32))

    def full_path():
        def vstep(t, T):
            b = 31 - t
            trial = T | (jnp.int32(1) << b)
            cnt = jnp.sum((key >= (trial ^ _INT_MIN)).astype(jnp.int32))
            return jnp.where(cnt >= K, trial, T)

        T = jax.lax.fori_loop(0, 32, vstep, jnp.int32(0))
        kth_f = T ^ _INT_MIN
        return kth_f, jnp.sum((key > kth_f).astype(jnp.int32))

    kth, c_gt = jax.lax.cond(
        c_gt_cand < K, lambda: (kth_cand, c_gt_cand), full_path)

    eq = key == kth
    t_eq = jnp.sum(eq.astype(jnp.int32))
    need = K - c_gt

    def simple_keep():
        return jnp.where(key >= kth, x, jnp.float32(-jnp.inf))

    def tie_keep():
        row = jax.lax.broadcasted_iota(jnp.int32, (R, C), 0)
        col = jax.lax.broadcasted_iota(jnp.int32, (R, C), 1)
        idx = row * C + col

        def istep(t, res):
            b = 19 - t
            trial = res | (jnp.int32(1) << b)
            cnt = jnp.sum((eq & (idx < trial)).astype(jnp.int32))
            return jnp.where(cnt < need, trial, res)

        res = jax.lax.fori_loop(0, 20, istep, jnp.int32(0))
        keep = (key > kth) | (eq & (idx <= res))
        return jnp.where(keep, x, jnp.float32(-jnp.inf))

    o_ref[...] = jax.lax.cond(need == t_eq, simple_keep, tie_keep)


def kernel(logits, position_ids):
    del position_ids
    x = logits.reshape(R, C)
    out = pl.pallas_call(
        _body,
        out_shape=jax.ShapeDtypeStruct((R, C), jnp.float32),
    )(x)
    return out.reshape(1, N)


# speedup vs baseline: 9.2595x; 1.5851x over previous
import jax
import jax.numpy as jnp
from jax.experimental import pallas as pl
from jax.experimental.pallas import tpu as pltpu

K = 50
N = 1_000_000
W = 8192
NFULL = N // W
TAIL = N - NFULL * W


def _key_of(x):
    ib = jax.lax.bitcast_convert_type(x, jnp.int32)
    return ib ^ ((ib >> 31) & jnp.int32(0x7FFFFFFF))


def _body(x_ref, o_ref):
    _INT_MIN = jnp.int32(-(2**31))
    zeros_w = jnp.zeros((1, W), jnp.int32)

    def fold(i, cs):
        v = x_ref[:, pl.ds(i * W, W)]
        c0, c1 = cs
        m = jnp.maximum(c0, v)
        v = jnp.minimum(c0, v)
        return (m, jnp.maximum(c1, v))

    neg = jnp.full((1, W), -jnp.inf, jnp.float32)
    c0, c1 = jax.lax.fori_loop(0, NFULL, fold, (neg, neg))
    tail = x_ref[:, pl.ds(NFULL * W, TAIL)]
    ktail = _key_of(tail)
    kcand = _key_of(jnp.concatenate([c0, c1], axis=1))

    def cstep(t, T):
        b = 31 - t
        trial = T | (jnp.int32(1) << b)
        thr = trial ^ _INT_MIN
        cnt = (jnp.sum((kcand >= thr).astype(jnp.int32))
               + jnp.sum((ktail >= thr).astype(jnp.int32)))
        return jnp.where(cnt >= K, trial, T)

    Tc = jax.lax.fori_loop(0, 32, cstep, jnp.int32(0))
    kth_cand = Tc ^ _INT_MIN

    def count_vs(kth):
        def step(i, accs):
            g, e = accs
            k = _key_of(x_ref[:, pl.ds(i * W, W)])
            g = g + (k > kth).astype(jnp.int32)
            e = e + (k == kth).astype(jnp.int32)
            return (g, e)

        g, e = jax.lax.fori_loop(0, NFULL, step, (zeros_w, zeros_w))
        c_gt = jnp.sum(g) + jnp.sum((ktail > kth).astype(jnp.int32))
        t_eq = jnp.sum(e) + jnp.sum((ktail == kth).astype(jnp.int32))
        return c_gt, t_eq

    c_gt_cand, t_eq_cand = count_vs(kth_cand)

    def full_path():
        def vstep(t, T):
            b = 31 - t
            trial = T | (jnp.int32(1) << b)
            thr = trial ^ _INT_MIN

            def cstep_f(i, a):
                k = _key_of(x_ref[:, pl.ds(i * W, W)])
                return a + (k >= thr).astype(jnp.int32)

            a = jax.lax.fori_loop(0, NFULL, cstep_f, zeros_w)
            cnt = jnp.sum(a) + jnp.sum((ktail >= thr).astype(jnp.int32))
            return jnp.where(cnt >= K, trial, T)

        T = jax.lax.fori_loop(0, 32, vstep, jnp.int32(0))
        kth_f = T ^ _INT_MIN
        c_gt_f, t_eq_f = count_vs(kth_f)
        return kth_f, c_gt_f, t_eq_f

    kth, c_gt, t_eq = jax.lax.cond(
        c_gt_cand < K,
        lambda: (kth_cand, c_gt_cand, t_eq_cand),
        full_path)
    need = K - c_gt

    def no_tie():
        return jnp.int32(N)

    def tie_path():
        def istep(t, res):
            b = 19 - t
            trial = res | (jnp.int32(1) << b)

            def cstep_i(i, a):
                k = _key_of(x_ref[:, pl.ds(i * W, W)])
                idx = jax.lax.broadcasted_iota(jnp.int32, (1, W), 1) + i * W
                return a + ((k == kth) & (idx < trial)).astype(jnp.int32)

            a = jax.lax.fori_loop(0, NFULL, cstep_i, zeros_w)
            idx_t = (jax.lax.broadcasted_iota(jnp.int32, (1, TAIL), 1)
                     + NFULL * W)
            cnt = (jnp.sum(a)
                   + jnp.sum(((ktail == kth) & (idx_t < trial))
                             .astype(jnp.int32)))
            return jnp.where(cnt < need, trial, res)

        return jax.lax.fori_loop(0, 20, istep, jnp.int32(0))

    res = jax.lax.cond(need == t_eq, no_tie, tie_path)

    def wstep(i, carry):
        v = x_ref[:, pl.ds(i * W, W)]
        k = _key_of(v)
        idx = jax.lax.broadcasted_iota(jnp.int32, (1, W), 1) + i * W
        keep = (k > kth) | ((k == kth) & (idx <= res))
        o_ref[:, pl.ds(i * W, W)] = jnp.where(keep, v, jnp.float32(-jnp.inf))
        return carry

    jax.lax.fori_loop(0, NFULL, wstep, jnp.int32(0))
    idx_t = jax.lax.broadcasted_iota(jnp.int32, (1, TAIL), 1) + NFULL * W
    keep_t = (ktail > kth) | ((ktail == kth) & (idx_t <= res))
    o_ref[:, pl.ds(NFULL * W, TAIL)] = jnp.where(
        keep_t, tail, jnp.float32(-jnp.inf))


def kernel(logits, position_ids):
    del position_ids
    return pl.pallas_call(
        _body,
        out_shape=jax.ShapeDtypeStruct((1, N), jnp.float32),
    )(logits)


# speedup vs baseline: 10.3132x; 1.1138x over previous
import jax
import jax.numpy as jnp
from jax.experimental import pallas as pl
from jax.experimental.pallas import tpu as pltpu

K = 50
N = 1_000_000
W = 8192
NFULL = N // W
TAIL = N - NFULL * W
UN = 4
NU = NFULL // UN
REM = NFULL - NU * UN


def _key_of(x):
    ib = jax.lax.bitcast_convert_type(x, jnp.int32)
    return ib ^ ((ib >> 31) & jnp.int32(0x7FFFFFFF))


def _top2_merge(c0, c1, first, second):
    t0 = jnp.maximum(c0, first)
    t1 = jnp.maximum(jnp.minimum(c0, first), jnp.maximum(c1, second))
    return t0, t1


def _top2_of4(v0, v1, v2, v3):
    m1 = jnp.maximum(v0, v1); n1 = jnp.minimum(v0, v1)
    m2 = jnp.maximum(v2, v3); n2 = jnp.minimum(v2, v3)
    first = jnp.maximum(m1, m2)
    second = jnp.maximum(jnp.minimum(m1, m2), jnp.maximum(n1, n2))
    return first, second


def _body(x_ref, o_ref):
    _INT_MIN = jnp.int32(-(2**31))
    zeros_w = jnp.zeros((1, W), jnp.int32)

    def fold(i, cs):
        base = i * (UN * W)
        v = [x_ref[:, pl.ds(base + j * W, W)] for j in range(UN)]
        first, second = _top2_of4(v[0], v[1], v[2], v[3])
        return _top2_merge(cs[0], cs[1], first, second)

    neg = jnp.full((1, W), -jnp.inf, jnp.float32)
    c0, c1 = jax.lax.fori_loop(0, NU, fold, (neg, neg))
    vA = x_ref[:, pl.ds((NFULL - 2) * W, W)]
    vB = x_ref[:, pl.ds((NFULL - 1) * W, W)]
    c0, c1 = _top2_merge(c0, c1, jnp.maximum(vA, vB), jnp.minimum(vA, vB))
    tail = x_ref[:, pl.ds(NFULL * W, TAIL)]
    ktail = _key_of(tail)

    h = W // 2
    d0, d1 = _top2_of4(c0[:, :h], c1[:, :h], c0[:, h:], c1[:, h:])
    kcand = _key_of(jnp.concatenate([d0, d1], axis=1))

    def cstep(t, T):
        b = 31 - t
        trial = T | (jnp.int32(1) << b)
        thr = trial ^ _INT_MIN
        cnt = (jnp.sum((kcand >= thr).astype(jnp.int32))
               + jnp.sum((ktail >= thr).astype(jnp.int32)))
        return jnp.where(cnt >= K, trial, T)

    Tc = jax.lax.fori_loop(0, 32, cstep, jnp.int32(0))
    kth_cand = Tc ^ _INT_MIN

    def count_vs(kth):
        def step(i, accs):
            g, e = accs
            base = i * (UN * W)
            for j in range(UN):
                k = _key_of(x_ref[:, pl.ds(base + j * W, W)])
                g = g + (k > kth).astype(jnp.int32)
                e = e + (k == kth).astype(jnp.int32)
            return (g, e)

        g, e = jax.lax.fori_loop(0, NU, step, (zeros_w, zeros_w))
        for j in range(NFULL - REM, NFULL):
            k = _key_of(x_ref[:, pl.ds(j * W, W)])
            g = g + (k > kth).astype(jnp.int32)
            e = e + (k == kth).astype(jnp.int32)
        c_gt = jnp.sum(g) + jnp.sum((ktail > kth).astype(jnp.int32))
        t_eq = jnp.sum(e) + jnp.sum((ktail == kth).astype(jnp.int32))
        return c_gt, t_eq

    c_gt_cand, t_eq_cand = count_vs(kth_cand)

    def full_path():
        def vstep(t, T):
            b = 31 - t
            trial = T | (jnp.int32(1) << b)
            thr = trial ^ _INT_MIN

            def cstep_f(i, a):
                k = _key_of(x_ref[:, pl.ds(i * W, W)])
                return a + (k >= thr).astype(jnp.int32)

            a = jax.lax.fori_loop(0, NFULL, cstep_f, zeros_w)
            cnt = jnp.sum(a) + jnp.sum((ktail >= thr).astype(jnp.int32))
            return jnp.where(cnt >= K, trial, T)

        T = jax.lax.fori_loop(0, 32, vstep, jnp.int32(0))
        kth_f = T ^ _INT_MIN
        c_gt_f, t_eq_f = count_vs(kth_f)
        return kth_f, c_gt_f, t_eq_f

    kth, c_gt, t_eq = jax.lax.cond(
        c_gt_cand < K,
        lambda: (kth_cand, c_gt_cand, t_eq_cand),
        full_path)
    need = K - c_gt

    def no_tie():
        return jnp.int32(N)

    def tie_path():
        def istep(t, res):
            b = 19 - t
            trial = res | (jnp.int32(1) << b)

            def cstep_i(i, a):
                k = _key_of(x_ref[:, pl.ds(i * W, W)])
                idx = jax.lax.broadcasted_iota(jnp.int32, (1, W), 1) + i * W
                return a + ((k == kth) & (idx < trial)).astype(jnp.int32)

            a = jax.lax.fori_loop(0, NFULL, cstep_i, zeros_w)
            idx_t = (jax.lax.broadcasted_iota(jnp.int32, (1, TAIL), 1)
                     + NFULL * W)
            cnt = (jnp.sum(a)
                   + jnp.sum(((ktail == kth) & (idx_t < trial))
                             .astype(jnp.int32)))
            return jnp.where(cnt < need, trial, res)

        return jax.lax.fori_loop(0, 20, istep, jnp.int32(0))

    res = jax.lax.cond(need == t_eq, no_tie, tie_path)

    iota_w = jax.lax.broadcasted_iota(jnp.int32, (1, W), 1)

    def write_one(c):
        v = x_ref[:, pl.ds(c * W, W)]
        k = _key_of(v)
        idx = iota_w + c * W
        keep = (k > kth) | ((k == kth) & (idx <= res))
        o_ref[:, pl.ds(c * W, W)] = jnp.where(keep, v, jnp.float32(-jnp.inf))

    def wstep(i, carry):
        base = i * UN
        v = [x_ref[:, pl.ds((base + j) * W, W)] for j in range(UN)]
        for j in range(UN):
            k = _key_of(v[j])
            idx = iota_w + (base + j) * W
            keep = (k > kth) | ((k == kth) & (idx <= res))
            o_ref[:, pl.ds((base + j) * W, W)] = jnp.where(
                keep, v[j], jnp.float32(-jnp.inf))
        return carry

    jax.lax.fori_loop(0, NU, wstep, jnp.int32(0))
    for c in range(NFULL - REM, NFULL):
        write_one(c)
    idx_t = jax.lax.broadcasted_iota(jnp.int32, (1, TAIL), 1) + NFULL * W
    keep_t = (ktail > kth) | ((ktail == kth) & (idx_t <= res))
    o_ref[:, pl.ds(NFULL * W, TAIL)] = jnp.where(
        keep_t, tail, jnp.float32(-jnp.inf))


def kernel(logits, position_ids):
    del position_ids
    return pl.pallas_call(
        _body,
        out_shape=jax.ShapeDtypeStruct((1, N), jnp.float32),
    )(logits)


# speedup vs baseline: 10.3136x; 1.0000x over previous
import jax
import jax.numpy as jnp
from jax.experimental import pallas as pl
from jax.experimental.pallas import tpu as pltpu

K = 50
N = 1_000_000
W = 8192
NFULL = N // W
TAIL = N - NFULL * W
UN = 4
NU = NFULL // UN
REM = NFULL - NU * UN


def _key_of(x):
    ib = jax.lax.bitcast_convert_type(x, jnp.int32)
    return ib ^ ((ib >> 31) & jnp.int32(0x7FFFFFFF))


def _top2_merge(c0, c1, first, second):
    t0 = jnp.maximum(c0, first)
    t1 = jnp.maximum(jnp.minimum(c0, first), jnp.maximum(c1, second))
    return t0, t1


def _top2_of4(v0, v1, v2, v3):
    m1 = jnp.maximum(v0, v1); n1 = jnp.minimum(v0, v1)
    m2 = jnp.maximum(v2, v3); n2 = jnp.minimum(v2, v3)
    first = jnp.maximum(m1, m2)
    second = jnp.maximum(jnp.minimum(m1, m2), jnp.maximum(n1, n2))
    return first, second


def _body(x_ref, o_ref):
    _INT_MIN = jnp.int32(-(2**31))
    zeros_w = jnp.zeros((1, W), jnp.int32)

    def fold(i, cs):
        base = pl.multiple_of(i * (UN * W), W)
        v = [x_ref[:, pl.ds(base + j * W, W)] for j in range(UN)]
        first, second = _top2_of4(v[0], v[1], v[2], v[3])
        return _top2_merge(cs[0], cs[1], first, second)

    neg = jnp.full((1, W), -jnp.inf, jnp.float32)
    c0, c1 = jax.lax.fori_loop(0, NU, fold, (neg, neg))
    vA = x_ref[:, pl.ds((NFULL - 2) * W, W)]
    vB = x_ref[:, pl.ds((NFULL - 1) * W, W)]
    c0, c1 = _top2_merge(c0, c1, jnp.maximum(vA, vB), jnp.minimum(vA, vB))
    tail = x_ref[:, pl.ds(NFULL * W, TAIL)]
    ktail = _key_of(tail)

    h = W // 2
    d0, d1 = _top2_of4(c0[:, :h], c1[:, :h], c0[:, h:], c1[:, h:])
    kcand = _key_of(jnp.concatenate([d0, d1], axis=1))

    def cstep(t, T):
        b = 31 - t
        trial = T | (jnp.int32(1) << b)
        thr = trial ^ _INT_MIN
        cnt = (jnp.sum((kcand >= thr).astype(jnp.int32))
               + jnp.sum((ktail >= thr).astype(jnp.int32)))
        return jnp.where(cnt >= K, trial, T)

    Tc = jax.lax.fori_loop(0, 32, cstep, jnp.int32(0))
    kth_cand = Tc ^ _INT_MIN

    def count_vs(kth):
        def step(i, accs):
            g, e = accs
            base = pl.multiple_of(i * (UN * W), W)
            for j in range(UN):
                k = _key_of(x_ref[:, pl.ds(base + j * W, W)])
                g = g + (k > kth).astype(jnp.int32)
                e = e + (k == kth).astype(jnp.int32)
            return (g, e)

        g, e = jax.lax.fori_loop(0, NU, step, (zeros_w, zeros_w))
        for j in range(NFULL - REM, NFULL):
            k = _key_of(x_ref[:, pl.ds(j * W, W)])
            g = g + (k > kth).astype(jnp.int32)
            e = e + (k == kth).astype(jnp.int32)
        c_gt = jnp.sum(g) + jnp.sum((ktail > kth).astype(jnp.int32))
        t_eq = jnp.sum(e) + jnp.sum((ktail == kth).astype(jnp.int32))
        return c_gt, t_eq

    c_gt_cand, t_eq_cand = count_vs(kth_cand)

    def full_path():
        def vstep(t, T):
            b = 31 - t
            trial = T | (jnp.int32(1) << b)
            thr = trial ^ _INT_MIN

            def cstep_f(i, a):
                k = _key_of(x_ref[:, pl.ds(pl.multiple_of(i * W, W), W)])
                return a + (k >= thr).astype(jnp.int32)

            a = jax.lax.fori_loop(0, NFULL, cstep_f, zeros_w)
            cnt = jnp.sum(a) + jnp.sum((ktail >= thr).astype(jnp.int32))
            return jnp.where(cnt >= K, trial, T)

        T = jax.lax.fori_loop(0, 32, vstep, jnp.int32(0))
        kth_f = T ^ _INT_MIN
        c_gt_f, t_eq_f = count_vs(kth_f)
        return kth_f, c_gt_f, t_eq_f

    kth, c_gt, t_eq = jax.lax.cond(
        c_gt_cand < K,
        lambda: (kth_cand, c_gt_cand, t_eq_cand),
        full_path)
    need = K - c_gt

    def no_tie():
        return jnp.int32(N)

    def tie_path():
        def istep(t, res):
            b = 19 - t
            trial = res | (jnp.int32(1) << b)

            def cstep_i(i, a):
                k = _key_of(x_ref[:, pl.ds(pl.multiple_of(i * W, W), W)])
                idx = jax.lax.broadcasted_iota(jnp.int32, (1, W), 1) + i * W
                return a + ((k == kth) & (idx < trial)).astype(jnp.int32)

            a = jax.lax.fori_loop(0, NFULL, cstep_i, zeros_w)
            idx_t = (jax.lax.broadcasted_iota(jnp.int32, (1, TAIL), 1)
                     + NFULL * W)
            cnt = (jnp.sum(a)
                   + jnp.sum(((ktail == kth) & (idx_t < trial))
                             .astype(jnp.int32)))
            return jnp.where(cnt < need, trial, res)

        return jax.lax.fori_loop(0, 20, istep, jnp.int32(0))

    res = jax.lax.cond(need == t_eq, no_tie, tie_path)

    iota_w = jax.lax.broadcasted_iota(jnp.int32, (1, W), 1)

    def write_one(c):
        v = x_ref[:, pl.ds(c * W, W)]
        k = _key_of(v)
        idx = iota_w + c * W
        keep = (k > kth) | ((k == kth) & (idx <= res))
        o_ref[:, pl.ds(c * W, W)] = jnp.where(keep, v, jnp.float32(-jnp.inf))

    def wstep(i, carry):
        base = i * UN
        off = pl.multiple_of(base * W, W)
        v = [x_ref[:, pl.ds(off + j * W, W)] for j in range(UN)]
        for j in range(UN):
            k = _key_of(v[j])
            idx = iota_w + (base + j) * W
            keep = (k > kth) | ((k == kth) & (idx <= res))
            o_ref[:, pl.ds(off + j * W, W)] = jnp.where(
                keep, v[j], jnp.float32(-jnp.inf))
        return carry

    jax.lax.fori_loop(0, NU, wstep, jnp.int32(0))
    for c in range(NFULL - REM, NFULL):
        write_one(c)
    idx_t = jax.lax.broadcasted_iota(jnp.int32, (1, TAIL), 1) + NFULL * W
    keep_t = (ktail > kth) | ((ktail == kth) & (idx_t <= res))
    o_ref[:, pl.ds(NFULL * W, TAIL)] = jnp.where(
        keep_t, tail, jnp.float32(-jnp.inf))


def kernel(logits, position_ids):
    del position_ids
    return pl.pallas_call(
        _body,
        out_shape=jax.ShapeDtypeStruct((1, N), jnp.float32),
    )(logits)
